# baseline (speedup 1.0000x reference)
import functools

import jax
import jax.numpy as jnp
from jax import lax
from jax.experimental import pallas as pl
from jax.experimental.pallas import tpu as pltpu

BN_EPS = 1e-5
VMEM_LIMIT = 100 * 1024 * 1024
PAD = 128


def _conv_pool_kernel(x_ref, wt_ref, b_ref, g_ref, s_ref,
                      psel_ref, stats_ref, x_sc, sel_sc, *, W):
    Cin, H = x_ref.shape[1], x_ref.shape[2]
    Cout = wt_ref.shape[0]
    HW = H * W

    x_sc[...] = x_ref[0].astype(jnp.bfloat16).reshape(Cin, HW)
    xb = x_sc[...]
    zp = jnp.zeros((Cin, PAD), jnp.bfloat16)
    xp = jnp.concatenate([zp, xb, zp], axis=1)

    lane = lax.broadcasted_iota(jnp.int32, (1, HW), 1)
    wpos = jnp.bitwise_and(lane, W - 1)
    mask_l = (wpos != 0).astype(jnp.bfloat16)
    mask_r = (wpos != W - 1).astype(jnp.bfloat16)

    taps = []
    for dy in range(3):
        for dx in range(3):
            s = (dy - 1) * W + (dx - 1)
            t = lax.slice(xp, (0, PAD + s), (Cin, PAD + s + HW))
            if dx == 0:
                t = t * mask_l
            elif dx == 2:
                t = t * mask_r
            taps.append(t)
    patch = jnp.concatenate(taps, axis=0)

    y = jnp.dot(wt_ref[...], patch,
                preferred_element_type=jnp.float32)
    y = y + b_ref[...]
    y = jnp.where(y > 0, y, jnp.exp(jnp.minimum(y, 0.0)) - 1.0)

    s1 = jnp.sum(y, axis=1, keepdims=True)
    s2 = jnp.sum(y * y, axis=1, keepdims=True)
    stats_ref[0] = jnp.concatenate([s1, s2], axis=1)

    yb = y.astype(jnp.bfloat16)
    zb = jnp.zeros((Cout, PAD), jnp.bfloat16)
    ybp = jnp.concatenate([yb, zb], axis=1)
    y1 = lax.slice(ybp, (0, 1), (Cout, 1 + HW))
    g = g_ref[...] >= 0
    selw = jnp.where(g, jnp.maximum(yb, y1), jnp.minimum(yb, y1))
    swp = jnp.concatenate([selw, zb], axis=1)
    s64 = lax.slice(swp, (0, W), (Cout, W + HW))
    sel = jnp.where(g, jnp.maximum(selw, s64), jnp.minimum(selw, s64))

    sel_sc[...] = sel.astype(jnp.float32).reshape(Cout, H, W)
    hsel = sel_sc[:, pl.ds(0, H // 2, 2), :]
    flat = hsel.reshape(Cout * (H // 2), W)
    ps = jnp.dot(flat, s_ref[...], preferred_element_type=jnp.float32)
    psel_ref[0] = ps.reshape(Cout, H // 2, W // 2)


def _affine_kernel(p_ref, sc_ref, sh_ref, o_ref):
    Cout = sc_ref.shape[0]
    o_ref[0] = (p_ref[0] * sc_ref[...].reshape(Cout, 1, 1)
                + sh_ref[...].reshape(Cout, 1, 1))


@jax.jit
def kernel(x_nchw, w_hwio, bias, gamma, beta):
    N, Cin, H, W = x_nchw.shape
    Cout = w_hwio.shape[-1]
    HW = H * W
    wt = jnp.transpose(w_hwio.reshape(9 * Cin, Cout)).astype(jnp.bfloat16)
    b2 = bias.reshape(Cout, 1).astype(jnp.float32)
    g2 = gamma.reshape(Cout, 1).astype(jnp.float32)

    smat = (jnp.arange(W)[:, None] == 2 * jnp.arange(W // 2)[None, :]
            ).astype(jnp.float32)

    psel, stats = pl.pallas_call(
        functools.partial(_conv_pool_kernel, W=W),
        out_shape=(
            jax.ShapeDtypeStruct((N, Cout, H // 2, W // 2), jnp.float32),
            jax.ShapeDtypeStruct((N, Cout, 2), jnp.float32),
        ),
        grid=(N,),
        in_specs=[
            pl.BlockSpec((1, Cin, H, W), lambda n: (n, 0, 0, 0)),
            pl.BlockSpec((Cout, 9 * Cin), lambda n: (0, 0)),
            pl.BlockSpec((Cout, 1), lambda n: (0, 0)),
            pl.BlockSpec((Cout, 1), lambda n: (0, 0)),
            pl.BlockSpec((W, W // 2), lambda n: (0, 0)),
        ],
        out_specs=(
            pl.BlockSpec((1, Cout, H // 2, W // 2), lambda n: (n, 0, 0, 0)),
            pl.BlockSpec((1, Cout, 2), lambda n: (n, 0, 0)),
        ),
        scratch_shapes=[
            pltpu.VMEM((Cin, HW), jnp.bfloat16),
            pltpu.VMEM((Cout, H, W), jnp.float32),
        ],
        compiler_params=pltpu.CompilerParams(
            dimension_semantics=("parallel",),
            vmem_limit_bytes=VMEM_LIMIT),
    )(x_nchw, wt, b2, g2, smat)

    cnt = float(N * H * W)
    mean = jnp.sum(stats[:, :, 0], axis=0) / cnt
    var = jnp.maximum(jnp.sum(stats[:, :, 1], axis=0) / cnt - mean * mean, 0.0)
    scale = gamma.reshape(-1) * lax.rsqrt(var + BN_EPS)
    shift = beta.reshape(-1) - mean * scale

    out = pl.pallas_call(
        _affine_kernel,
        out_shape=jax.ShapeDtypeStruct((N, Cout, H // 2, W // 2), jnp.float32),
        grid=(N,),
        in_specs=[
            pl.BlockSpec((1, Cout, H // 2, W // 2), lambda n: (n, 0, 0, 0)),
            pl.BlockSpec((Cout, 1), lambda n: (0, 0)),
            pl.BlockSpec((Cout, 1), lambda n: (0, 0)),
        ],
        out_specs=pl.BlockSpec((1, Cout, H // 2, W // 2),
                               lambda n: (n, 0, 0, 0)),
        compiler_params=pltpu.CompilerParams(
            dimension_semantics=("parallel",),
            vmem_limit_bytes=VMEM_LIMIT),
    )(psel, scale.reshape(Cout, 1), shift.reshape(Cout, 1))

    return out


# speedup vs baseline: 1.1318x; 1.1318x over previous
import functools

import jax
import jax.numpy as jnp
from jax import lax
from jax.experimental import pallas as pl
from jax.experimental.pallas import tpu as pltpu

BN_EPS = 1e-5
VMEM_LIMIT = 100 * 1024 * 1024
PAD = 128


def _conv_pool_kernel(x_ref, wt_ref, b_ref, g_ref, s_ref,
                      psel_ref, stats_ref, x_sc, sel_sc, *, W):
    Cin, H = x_ref.shape[1], x_ref.shape[2]
    Cout = wt_ref.shape[0]
    HW = H * W

    x_sc[:, 0:PAD] = jnp.zeros((Cin, PAD), jnp.bfloat16)
    x_sc[:, PAD + HW:] = jnp.zeros((Cin, PAD), jnp.bfloat16)
    x_sc[:, PAD:PAD + HW] = x_ref[0].astype(jnp.bfloat16).reshape(Cin, HW)
    xp = x_sc[...]

    lane = lax.broadcasted_iota(jnp.int32, (1, HW), 1)
    wpos = jnp.bitwise_and(lane, W - 1)
    mask_l = (wpos != 0).astype(jnp.bfloat16)
    mask_r = (wpos != W - 1).astype(jnp.bfloat16)

    taps = []
    for dy in range(3):
        for dx in range(3):
            s = (dy - 1) * W + (dx - 1)
            t = lax.slice(xp, (0, PAD + s), (Cin, PAD + s + HW))
            if dx == 0:
                t = t * mask_l
            elif dx == 2:
                t = t * mask_r
            taps.append(t)
    patch = jnp.concatenate(taps, axis=0)

    y = jnp.dot(wt_ref[...], patch,
                preferred_element_type=jnp.float32)
    y = y + b_ref[...]
    y = jnp.where(y > 0, y, jnp.exp(jnp.minimum(y, 0.0)) - 1.0)

    s1 = jnp.sum(y, axis=1, keepdims=True)
    s2 = jnp.sum(y * y, axis=1, keepdims=True)
    stats_ref[0] = jnp.concatenate([s1, s2], axis=1)

    yb = y.astype(jnp.bfloat16)
    zb = jnp.zeros((Cout, PAD), jnp.bfloat16)
    ybp = jnp.concatenate([yb, zb], axis=1)
    y1 = lax.slice(ybp, (0, 1), (Cout, 1 + HW))
    g = g_ref[...] >= 0
    selw = jnp.where(g, jnp.maximum(yb, y1), jnp.minimum(yb, y1))
    sel_sc[...] = selw.reshape(Cout, H // 2, 2 * W)
    a = sel_sc[:, :, 0:W]
    b = sel_sc[:, :, W:2 * W]
    g3 = g.reshape(Cout, 1, 1)
    hm = jnp.where(g3, jnp.maximum(a, b), jnp.minimum(a, b))
    flat = hm.reshape(Cout * (H // 2), W)
    ps = jnp.dot(flat, s_ref[...], preferred_element_type=jnp.float32)
    psel_ref[0] = ps.reshape(Cout, H // 2, W // 2)


def _affine_kernel(p_ref, sc_ref, sh_ref, o_ref):
    Cout, Hp, Wp = p_ref.shape[1], p_ref.shape[2], p_ref.shape[3]
    v = (p_ref[0] * sc_ref[...].reshape(Cout, 1, 1)
         + sh_ref[...].reshape(Cout, 1, 1))
    o_ref[0] = v.reshape(Cout, Hp * Wp)


@jax.jit
def kernel(x_nchw, w_hwio, bias, gamma, beta):
    N, Cin, H, W = x_nchw.shape
    Cout = w_hwio.shape[-1]
    HW = H * W
    wt = jnp.transpose(w_hwio.reshape(9 * Cin, Cout)).astype(jnp.bfloat16)
    b2 = bias.reshape(Cout, 1).astype(jnp.float32)
    g2 = gamma.reshape(Cout, 1).astype(jnp.float32)

    smat = (jnp.arange(W)[:, None] == 2 * jnp.arange(W // 2)[None, :]
            ).astype(jnp.bfloat16)

    psel, stats = pl.pallas_call(
        functools.partial(_conv_pool_kernel, W=W),
        out_shape=(
            jax.ShapeDtypeStruct((N, Cout, H // 2, W // 2), jnp.float32),
            jax.ShapeDtypeStruct((N, Cout, 2), jnp.float32),
        ),
        grid=(N,),
        in_specs=[
            pl.BlockSpec((1, Cin, H, W), lambda n: (n, 0, 0, 0)),
            pl.BlockSpec((Cout, 9 * Cin), lambda n: (0, 0)),
            pl.BlockSpec((Cout, 1), lambda n: (0, 0)),
            pl.BlockSpec((Cout, 1), lambda n: (0, 0)),
            pl.BlockSpec((W, W // 2), lambda n: (0, 0)),
        ],
        out_specs=(
            pl.BlockSpec((1, Cout, H // 2, W // 2), lambda n: (n, 0, 0, 0)),
            pl.BlockSpec((1, Cout, 2), lambda n: (n, 0, 0)),
        ),
        scratch_shapes=[
            pltpu.VMEM((Cin, HW + 2 * PAD), jnp.bfloat16),
            pltpu.VMEM((Cout, H // 2, 2 * W), jnp.bfloat16),
        ],
        compiler_params=pltpu.CompilerParams(
            dimension_semantics=("parallel",),
            vmem_limit_bytes=VMEM_LIMIT),
    )(x_nchw, wt, b2, g2, smat)

    cnt = float(N * H * W)
    mean = jnp.sum(stats[:, :, 0], axis=0) / cnt
    var = jnp.maximum(jnp.sum(stats[:, :, 1], axis=0) / cnt - mean * mean, 0.0)
    scale = gamma.reshape(-1) * lax.rsqrt(var + BN_EPS)
    shift = beta.reshape(-1) - mean * scale

    out = pl.pallas_call(
        _affine_kernel,
        out_shape=jax.ShapeDtypeStruct((N, Cout, HW // 4), jnp.float32),
        grid=(N,),
        in_specs=[
            pl.BlockSpec((1, Cout, H // 2, W // 2), lambda n: (n, 0, 0, 0)),
            pl.BlockSpec((Cout, 1), lambda n: (0, 0)),
            pl.BlockSpec((Cout, 1), lambda n: (0, 0)),
        ],
        out_specs=pl.BlockSpec((1, Cout, HW // 4), lambda n: (n, 0, 0)),
        compiler_params=pltpu.CompilerParams(
            dimension_semantics=("parallel",),
            vmem_limit_bytes=VMEM_LIMIT),
    )(psel, scale.reshape(Cout, 1), shift.reshape(Cout, 1))

    return out.reshape(N, Cout, H // 2, W // 2)


# speedup vs baseline: 1.1433x; 1.0101x over previous
import functools

import jax
import jax.numpy as jnp
from jax import lax
from jax.experimental import pallas as pl
from jax.experimental.pallas import tpu as pltpu

BN_EPS = 1e-5
VMEM_LIMIT = 100 * 1024 * 1024
PAD = 128


def _conv_pool_kernel(xt_ref, xb_ref, wt_ref, b_ref, g_ref, s_ref,
                      psel_ref, stats_ref, x_sc, sel_sc, *, W):
    Cin, H = xt_ref.shape[1], 2 * xt_ref.shape[2]
    Cout = wt_ref.shape[0]
    HW = H * W

    x_sc[:, 0:PAD] = jnp.zeros((Cin, PAD), jnp.bfloat16)
    x_sc[:, PAD + HW:] = jnp.zeros((Cin, PAD), jnp.bfloat16)
    half = HW // 2
    x_sc[:, PAD:PAD + half] = (
        xt_ref[0].astype(jnp.bfloat16).reshape(Cin, half))
    x_sc[:, PAD + half:PAD + HW] = (
        xb_ref[0].astype(jnp.bfloat16).reshape(Cin, half))
    xp = x_sc[...]

    lane = lax.broadcasted_iota(jnp.int32, (1, HW), 1)
    wpos = jnp.bitwise_and(lane, W - 1)
    mask_l = (wpos != 0).astype(jnp.bfloat16)
    mask_r = (wpos != W - 1).astype(jnp.bfloat16)

    taps = []
    for dy in range(3):
        for dx in range(3):
            s = (dy - 1) * W + (dx - 1)
            t = lax.slice(xp, (0, PAD + s), (Cin, PAD + s + HW))
            if dx == 0:
                t = t * mask_l
            elif dx == 2:
                t = t * mask_r
            taps.append(t)
    patch = jnp.concatenate(taps, axis=0)

    y = jnp.dot(wt_ref[...], patch,
                preferred_element_type=jnp.float32)
    y = y + b_ref[...]
    y = jnp.where(y > 0, y, jnp.exp(jnp.minimum(y, 0.0)) - 1.0)

    s1 = jnp.sum(y, axis=1, keepdims=True)
    s2 = jnp.sum(y * y, axis=1, keepdims=True)
    stats_ref[0] = jnp.concatenate([s1, s2], axis=1)

    yb = y.astype(jnp.bfloat16)
    zb = jnp.zeros((Cout, PAD), jnp.bfloat16)
    ybp = jnp.concatenate([yb, zb], axis=1)
    y1 = lax.slice(ybp, (0, 1), (Cout, 1 + HW))
    g = g_ref[...] >= 0
    selw = jnp.where(g, jnp.maximum(yb, y1), jnp.minimum(yb, y1))
    sel_sc[...] = selw.reshape(Cout, H // 2, 2 * W)
    a = sel_sc[:, :, 0:W]
    b = sel_sc[:, :, W:2 * W]
    g3 = g.reshape(Cout, 1, 1)
    hm = jnp.where(g3, jnp.maximum(a, b), jnp.minimum(a, b))
    flat = hm.reshape(Cout * (H // 2), W)
    ps = jnp.dot(flat, s_ref[...], preferred_element_type=jnp.float32)
    psel_ref[0] = ps.astype(jnp.bfloat16).reshape(Cout, H // 2, W // 2)


def _affine_kernel(p_ref, sc_ref, sh_ref, o_ref):
    Cout, Hp, Wp = p_ref.shape[1], p_ref.shape[2], p_ref.shape[3]
    v = (p_ref[0] * sc_ref[...].reshape(Cout, 1, 1)
         + sh_ref[...].reshape(Cout, 1, 1))
    o_ref[0] = v.reshape(Cout, Hp * Wp)


@jax.jit
def kernel(x_nchw, w_hwio, bias, gamma, beta):
    N, Cin, H, W = x_nchw.shape
    Cout = w_hwio.shape[-1]
    HW = H * W
    wt = jnp.transpose(w_hwio.reshape(9 * Cin, Cout)).astype(jnp.bfloat16)
    b2 = bias.reshape(Cout, 1).astype(jnp.float32)
    g2 = gamma.reshape(Cout, 1).astype(jnp.float32)

    smat = (jnp.arange(W)[:, None] == 2 * jnp.arange(W // 2)[None, :]
            ).astype(jnp.bfloat16)

    psel, stats = pl.pallas_call(
        functools.partial(_conv_pool_kernel, W=W),
        out_shape=(
            jax.ShapeDtypeStruct((N, Cout, H // 2, W // 2), jnp.bfloat16),
            jax.ShapeDtypeStruct((N, Cout, 2), jnp.float32),
        ),
        grid=(N,),
        in_specs=[
            pl.BlockSpec((1, Cin, H // 2, W), lambda n: (n, 0, 0, 0)),
            pl.BlockSpec((1, Cin, H // 2, W), lambda n: (n, 0, 1, 0)),
            pl.BlockSpec((Cout, 9 * Cin), lambda n: (0, 0)),
            pl.BlockSpec((Cout, 1), lambda n: (0, 0)),
            pl.BlockSpec((Cout, 1), lambda n: (0, 0)),
            pl.BlockSpec((W, W // 2), lambda n: (0, 0)),
        ],
        out_specs=(
            pl.BlockSpec((1, Cout, H // 2, W // 2), lambda n: (n, 0, 0, 0)),
            pl.BlockSpec((1, Cout, 2), lambda n: (n, 0, 0)),
        ),
        scratch_shapes=[
            pltpu.VMEM((Cin, HW + 2 * PAD), jnp.bfloat16),
            pltpu.VMEM((Cout, H // 2, 2 * W), jnp.bfloat16),
        ],
        compiler_params=pltpu.CompilerParams(
            dimension_semantics=("parallel",),
            vmem_limit_bytes=VMEM_LIMIT),
    )(x_nchw, x_nchw, wt, b2, g2, smat)

    cnt = float(N * H * W)
    mean = jnp.sum(stats[:, :, 0], axis=0) / cnt
    var = jnp.maximum(jnp.sum(stats[:, :, 1], axis=0) / cnt - mean * mean, 0.0)
    scale = gamma.reshape(-1) * lax.rsqrt(var + BN_EPS)
    shift = beta.reshape(-1) - mean * scale

    out = pl.pallas_call(
        _affine_kernel,
        out_shape=jax.ShapeDtypeStruct((N, Cout, HW // 4), jnp.float32),
        grid=(N,),
        in_specs=[
            pl.BlockSpec((1, Cout, H // 2, W // 2), lambda n: (n, 0, 0, 0)),
            pl.BlockSpec((Cout, 1), lambda n: (0, 0)),
            pl.BlockSpec((Cout, 1), lambda n: (0, 0)),
        ],
        out_specs=pl.BlockSpec((1, Cout, HW // 4), lambda n: (n, 0, 0)),
        compiler_params=pltpu.CompilerParams(
            dimension_semantics=("parallel",),
            vmem_limit_bytes=VMEM_LIMIT),
    )(psel, scale.reshape(Cout, 1), shift.reshape(Cout, 1))

    return out.reshape(N, Cout, H // 2, W // 2)


# speedup vs baseline: 1.2040x; 1.0531x over previous
import functools

import jax
import jax.numpy as jnp
from jax import lax
from jax.experimental import pallas as pl
from jax.experimental.pallas import tpu as pltpu

BN_EPS = 1e-5
VMEM_LIMIT = 100 * 1024 * 1024
PAD = 128


def _conv_pool_kernel(xt_ref, xb_ref, wt_ref, b_ref, g_ref, s_ref,
                      psel_ref, stats_ref, x_sc, sel_sc, *, W, IPS):
    Cin, H = xt_ref.shape[1], 2 * xt_ref.shape[2]
    Cout = wt_ref.shape[0]
    HW = H * W
    half = HW // 2

    lane = lax.broadcasted_iota(jnp.int32, (1, HW), 1)
    wpos = jnp.bitwise_and(lane, W - 1)
    mask_l = (wpos != 0).astype(jnp.bfloat16)
    mask_r = (wpos != W - 1).astype(jnp.bfloat16)
    g = g_ref[...] >= 0
    g3 = g.reshape(Cout, 1, 1)

    for img in range(IPS):
        x_sc[:, 0:PAD] = jnp.zeros((Cin, PAD), jnp.bfloat16)
        x_sc[:, PAD + HW:] = jnp.zeros((Cin, PAD), jnp.bfloat16)
        x_sc[:, PAD:PAD + half] = (
            xt_ref[img].astype(jnp.bfloat16).reshape(Cin, half))
        x_sc[:, PAD + half:PAD + HW] = (
            xb_ref[img].astype(jnp.bfloat16).reshape(Cin, half))
        xp = x_sc[...]

        taps = []
        for dy in range(3):
            for dx in range(3):
                s = (dy - 1) * W + (dx - 1)
                t = lax.slice(xp, (0, PAD + s), (Cin, PAD + s + HW))
                if dx == 0:
                    t = t * mask_l
                elif dx == 2:
                    t = t * mask_r
                taps.append(t)
        patch = jnp.concatenate(taps, axis=0)

        y = jnp.dot(wt_ref[...], patch,
                    preferred_element_type=jnp.float32)
        y = y + b_ref[...]
        y = jnp.where(y > 0, y, jnp.exp(jnp.minimum(y, 0.0)) - 1.0)

        s1 = jnp.sum(y, axis=1, keepdims=True)
        s2 = jnp.sum(y * y, axis=1, keepdims=True)
        stats_ref[img] = jnp.concatenate([s1, s2], axis=1)

        yb = y.astype(jnp.bfloat16)
        zb = jnp.zeros((Cout, PAD), jnp.bfloat16)
        ybp = jnp.concatenate([yb, zb], axis=1)
        y1 = lax.slice(ybp, (0, 1), (Cout, 1 + HW))
        selw = jnp.where(g, jnp.maximum(yb, y1), jnp.minimum(yb, y1))
        sel_sc[...] = selw.reshape(Cout, H // 2, 2 * W)
        a = sel_sc[:, :, 0:W]
        b = sel_sc[:, :, W:2 * W]
        hm = jnp.where(g3, jnp.maximum(a, b), jnp.minimum(a, b))
        flat = hm.reshape(Cout * (H // 2), W)
        ps = jnp.dot(flat, s_ref[...], preferred_element_type=jnp.float32)
        psel_ref[img] = ps.astype(jnp.bfloat16).reshape(Cout, H // 2, W // 2)


def _affine_kernel(p_ref, sc_ref, sh_ref, o_ref):
    IPS, Cout, Hp, Wp = p_ref.shape
    for img in range(IPS):
        v = (p_ref[img] * sc_ref[...].reshape(Cout, 1, 1)
             + sh_ref[...].reshape(Cout, 1, 1))
        o_ref[img] = v.reshape(Cout, Hp * Wp)


@jax.jit
def kernel(x_nchw, w_hwio, bias, gamma, beta):
    N, Cin, H, W = x_nchw.shape
    Cout = w_hwio.shape[-1]
    HW = H * W
    wt = jnp.transpose(w_hwio.reshape(9 * Cin, Cout)).astype(jnp.bfloat16)
    b2 = bias.reshape(Cout, 1).astype(jnp.float32)
    g2 = gamma.reshape(Cout, 1).astype(jnp.float32)

    smat = (jnp.arange(W)[:, None] == 2 * jnp.arange(W // 2)[None, :]
            ).astype(jnp.bfloat16)

    IPS = 2 if N % 2 == 0 else 1
    psel, stats = pl.pallas_call(
        functools.partial(_conv_pool_kernel, W=W, IPS=IPS),
        out_shape=(
            jax.ShapeDtypeStruct((N, Cout, H // 2, W // 2), jnp.bfloat16),
            jax.ShapeDtypeStruct((N, Cout, 2), jnp.float32),
        ),
        grid=(N // IPS,),
        in_specs=[
            pl.BlockSpec((IPS, Cin, H // 2, W), lambda n: (n, 0, 0, 0)),
            pl.BlockSpec((IPS, Cin, H // 2, W), lambda n: (n, 0, 1, 0)),
            pl.BlockSpec((Cout, 9 * Cin), lambda n: (0, 0)),
            pl.BlockSpec((Cout, 1), lambda n: (0, 0)),
            pl.BlockSpec((Cout, 1), lambda n: (0, 0)),
            pl.BlockSpec((W, W // 2), lambda n: (0, 0)),
        ],
        out_specs=(
            pl.BlockSpec((IPS, Cout, H // 2, W // 2),
                         lambda n: (n, 0, 0, 0)),
            pl.BlockSpec((IPS, Cout, 2), lambda n: (n, 0, 0)),
        ),
        scratch_shapes=[
            pltpu.VMEM((Cin, HW + 2 * PAD), jnp.bfloat16),
            pltpu.VMEM((Cout, H // 2, 2 * W), jnp.bfloat16),
        ],
        compiler_params=pltpu.CompilerParams(
            dimension_semantics=("parallel",),
            vmem_limit_bytes=VMEM_LIMIT),
    )(x_nchw, x_nchw, wt, b2, g2, smat)

    cnt = float(N * H * W)
    mean = jnp.sum(stats[:, :, 0], axis=0) / cnt
    var = jnp.maximum(jnp.sum(stats[:, :, 1], axis=0) / cnt - mean * mean, 0.0)
    scale = gamma.reshape(-1) * lax.rsqrt(var + BN_EPS)
    shift = beta.reshape(-1) - mean * scale

    out = pl.pallas_call(
        _affine_kernel,
        out_shape=jax.ShapeDtypeStruct((N, Cout, HW // 4), jnp.float32),
        grid=(N // IPS,),
        in_specs=[
            pl.BlockSpec((IPS, Cout, H // 2, W // 2),
                         lambda n: (n, 0, 0, 0)),
            pl.BlockSpec((Cout, 1), lambda n: (0, 0)),
            pl.BlockSpec((Cout, 1), lambda n: (0, 0)),
        ],
        out_specs=pl.BlockSpec((IPS, Cout, HW // 4), lambda n: (n, 0, 0)),
        compiler_params=pltpu.CompilerParams(
            dimension_semantics=("parallel",),
            vmem_limit_bytes=VMEM_LIMIT),
    )(psel, scale.reshape(Cout, 1), shift.reshape(Cout, 1))

    return out.reshape(N, Cout, H // 2, W // 2)
